# baseline (speedup 1.0000x reference)
import functools

import jax
import jax.numpy as jnp
from jax import lax
from jax.experimental import pallas as pl
from jax.experimental.pallas import tpu as pltpu
from jax.experimental.pallas import tpu_sc as plsc

V = 10000
E = 320000
D = 128

NC = 2
NS = 16
NW = NC * NS

CHUNK = 128
NCH = E // CHUNK
NSLOT = 80
NBUF = 2
VP = 10240
RPT = VP // NS

_mesh = plsc.VectorSubcoreMesh(core_axis_name="c", subcore_axis_name="s")


@functools.partial(
    pl.kernel,
    out_type=jax.ShapeDtypeStruct((NC, VP, D), jnp.float32),
    mesh=_mesh,
    scratch_types=(
        [pltpu.VMEM((CHUNK, D), jnp.float32) for _ in range(NBUF)]
        + [pltpu.VMEM((2, CHUNK), jnp.int32) for _ in range(NBUF)]
        + [pltpu.VMEM_SHARED((VP, D), jnp.float32)]
        + [pltpu.SemaphoreType.DMA for _ in range(2 * NBUF)]
    ),
)
def _scatter_add_sc(x_hbm, idx_hbm, out_hbm, *scratch):
    xb = scratch[0:NBUF]
    ib = scratch[NBUF:2 * NBUF]
    msg_sh = scratch[2 * NBUF]
    lsem = scratch[2 * NBUF + 1:2 * NBUF + 1 + NBUF]
    ssem = scratch[2 * NBUF + 1 + NBUF:2 * NBUF + 1 + 2 * NBUF]

    c = lax.axis_index("c")
    s = lax.axis_index("s")
    w = c * NS + s

    def issue_loads(b, chid):
        pltpu.async_copy(x_hbm.at[chid], xb[b], lsem[b])
        pltpu.async_copy(idx_hbm.at[chid], ib[b], lsem[b])

    def wait_loads(b):
        pltpu.make_async_copy(x_hbm.at[0], xb[b], lsem[b]).wait()
        pltpu.make_async_copy(idx_hbm.at[0], ib[b], lsem[b]).wait()

    def drain_scatters(b):
        pltpu.make_async_copy(xb[b], msg_sh.at[ib[b].at[0]], ssem[b]).wait()
        pltpu.make_async_copy(xb[b], msg_sh.at[ib[b].at[1]], ssem[b]).wait()

    issue_loads(0, w)

    zrow = jnp.zeros((16,), jnp.float32)

    def zfill(i, carry):
        xb[1][i // 8, pl.ds((i % 8) * 16, 16)] = zrow
        return carry

    lax.fori_loop(0, CHUNK * (D // 16), zfill, 0)
    for r in range(RPT // CHUNK):
        pltpu.sync_copy(xb[1], msg_sh.at[pl.ds(s * RPT + r * CHUNK, CHUNK)])
    plsc.subcore_barrier()

    def group(g, carry):
        for p in range(NBUF):
            j = NBUF * g + p
            chid = w + NW * j
            q = (p + 1) % NBUF

            @pl.when(chid < NCH)
            def _():
                wait_loads(p)
                pltpu.async_copy(xb[p], msg_sh.at[ib[p].at[0]], ssem[p], add=True)
                pltpu.async_copy(xb[p], msg_sh.at[ib[p].at[1]], ssem[p], add=True)

            @pl.when(jnp.logical_and(chid >= NW, chid - NW < NCH))
            def _():
                drain_scatters(q)

            @pl.when(chid + NW < NCH)
            def _():
                issue_loads(q, chid + NW)
        return carry

    lax.fori_loop(0, NSLOT // NBUF, group, 0)
    plsc.subcore_barrier()

    pltpu.sync_copy(msg_sh.at[pl.ds(s * RPT, RPT)],
                    out_hbm.at[c, pl.ds(s * RPT, RPT)])


BLK = 1000


def _lstm_body(p_ref, h_ref, c_ref, wih_ref, whh_ref, b_ref, ho_ref, co_ref):
    msg = p_ref[0] + p_ref[1]
    h = h_ref[...]
    gates = lax.dot_general(msg, wih_ref[...], (((1,), (1,)), ((), ())),
                            preferred_element_type=jnp.float32)
    gates = gates + lax.dot_general(h, whh_ref[...], (((1,), (1,)), ((), ())),
                                    preferred_element_type=jnp.float32)
    gates = gates + b_ref[...]
    i = jax.nn.sigmoid(gates[:, 0 * D:1 * D])
    f = jax.nn.sigmoid(gates[:, 1 * D:2 * D])
    g = jnp.tanh(gates[:, 2 * D:3 * D])
    o = jax.nn.sigmoid(gates[:, 3 * D:4 * D])
    c_new = f * c_ref[...] + i * g
    ho_ref[...] = o * jnp.tanh(c_new)
    co_ref[...] = c_new


_lstm_call = pl.pallas_call(
    _lstm_body,
    grid=(V // BLK,),
    in_specs=[
        pl.BlockSpec((NC, BLK, D), lambda i: (0, i, 0)),
        pl.BlockSpec((BLK, D), lambda i: (i, 0)),
        pl.BlockSpec((BLK, D), lambda i: (i, 0)),
        pl.BlockSpec((4 * D, D), lambda i: (0, 0)),
        pl.BlockSpec((4 * D, D), lambda i: (0, 0)),
        pl.BlockSpec((1, 4 * D), lambda i: (0, 0)),
    ],
    out_specs=[
        pl.BlockSpec((BLK, D), lambda i: (i, 0)),
        pl.BlockSpec((BLK, D), lambda i: (i, 0)),
    ],
    out_shape=[
        jax.ShapeDtypeStruct((V, D), jnp.float32),
        jax.ShapeDtypeStruct((V, D), jnp.float32),
    ],
)


@jax.jit
def kernel(x_e, edge_index, h_v, c_v, v_batch, W_ih, W_hh, b_ih, b_hh):
    del v_batch
    x3 = x_e.reshape(NCH, CHUNK, D)
    idx3 = edge_index.astype(jnp.int32).reshape(2, NCH, CHUNK).transpose(1, 0, 2)
    partials = _scatter_add_sc(x3, idx3)
    bias = (b_ih + b_hh).reshape(1, 4 * D)
    h_new, c_new = _lstm_call(partials, h_v, c_v, W_ih, W_hh, bias)
    return (h_new, c_new)
